# baseline (device time: 16815 ns/iter reference)
import jax
import jax.numpy as jnp
from jax import lax
from jax.experimental import pallas as pl
from jax.experimental.pallas import tpu as pltpu

N_DEV = 4
EPS = 1e-5
N_GLOBAL = 2048.0


def kernel(x, gamma):
    m, n_per = x.shape
    gamma2 = gamma.reshape(1, n_per)

    def body(x_ref, g_ref, out_ref, send_buf, comm_ref, send_sems, recv_sems):
        my = lax.axis_index("i")

        barrier_sem = pltpu.get_barrier_semaphore()
        for e in range(1, N_DEV):
            pl.semaphore_signal(
                barrier_sem, inc=1,
                device_id=((my + e) % N_DEV,),
                device_id_type=pl.DeviceIdType.MESH,
            )

        xv = x_ref[:, :]
        partial = jnp.sum(xv * xv, axis=1, keepdims=True)
        send_buf[:, :] = partial

        pl.semaphore_wait(barrier_sem, N_DEV - 1)

        sends = []
        for e in range(1, N_DEV):
            d = N_DEV - e
            rdma = pltpu.make_async_remote_copy(
                src_ref=send_buf,
                dst_ref=comm_ref.at[d],
                send_sem=send_sems.at[e - 1],
                recv_sem=recv_sems.at[d],
                device_id=((my + e) % N_DEV,),
                device_id_type=pl.DeviceIdType.MESH,
            )
            rdma.start()
            sends.append(rdma)

        total = partial
        for d in range(1, N_DEV):
            recv = pltpu.make_async_remote_copy(
                src_ref=send_buf,
                dst_ref=comm_ref.at[d],
                send_sem=send_sems.at[0],
                recv_sem=recv_sems.at[d],
                device_id=(my,),
                device_id_type=pl.DeviceIdType.MESH,
            )
            recv.wait_recv()
            total = total + comm_ref[d]

        inv_rms = lax.rsqrt(total * jnp.float32(1.0 / N_GLOBAL) + jnp.float32(EPS))
        out_ref[:, :] = g_ref[:, :] * xv * inv_rms

        for rdma in sends:
            rdma.wait_send()

    return pl.pallas_call(
        body,
        out_shape=jax.ShapeDtypeStruct((m, n_per), jnp.float32),
        in_specs=[
            pl.BlockSpec(memory_space=pltpu.VMEM),
            pl.BlockSpec(memory_space=pltpu.VMEM),
        ],
        out_specs=pl.BlockSpec(memory_space=pltpu.VMEM),
        scratch_shapes=[
            pltpu.VMEM((m, 1), jnp.float32),
            pltpu.VMEM((N_DEV, m, 1), jnp.float32),
            pltpu.SemaphoreType.DMA((N_DEV - 1,)),
            pltpu.SemaphoreType.DMA((N_DEV,)),
        ],
        compiler_params=pltpu.CompilerParams(collective_id=0),
    )(x, gamma2)


# device time: 8567 ns/iter; 1.9628x vs baseline; 1.9628x over previous
import jax
import jax.numpy as jnp
from jax import lax
from jax.experimental import pallas as pl
from jax.experimental.pallas import tpu as pltpu

N_DEV = 4
EPS = 1e-5
N_GLOBAL = 2048.0


def kernel(x, gamma):
    m, n_per = x.shape
    gamma2 = gamma.reshape(1, n_per)

    def body(x_ref, g_ref, out_ref, send_buf, comm_ref, send_sems, recv_sems):
        my = lax.axis_index("i")

        barrier_sem = pltpu.get_barrier_semaphore()
        for e in range(1, N_DEV):
            pl.semaphore_signal(
                barrier_sem, inc=1,
                device_id=((my + e) % N_DEV,),
                device_id_type=pl.DeviceIdType.MESH,
            )

        xv = x_ref[:, :]
        partial = jnp.sum(xv * xv, axis=1, keepdims=True)
        send_buf[:, :] = lax.transpose(partial, (1, 0))

        pl.semaphore_wait(barrier_sem, N_DEV - 1)

        sends = []
        for e in range(1, N_DEV):
            d = N_DEV - e
            rdma = pltpu.make_async_remote_copy(
                src_ref=send_buf,
                dst_ref=comm_ref.at[d],
                send_sem=send_sems.at[e - 1],
                recv_sem=recv_sems.at[d],
                device_id=((my + e) % N_DEV,),
                device_id_type=pl.DeviceIdType.MESH,
            )
            rdma.start()
            sends.append(rdma)

        y = g_ref[:, :] * xv

        total = send_buf[:, :]
        for d in range(1, N_DEV):
            recv = pltpu.make_async_remote_copy(
                src_ref=send_buf,
                dst_ref=comm_ref.at[d],
                send_sem=send_sems.at[0],
                recv_sem=recv_sems.at[d],
                device_id=(my,),
                device_id_type=pl.DeviceIdType.MESH,
            )
            recv.wait_recv()
            total = total + comm_ref[d]

        inv_rms = lax.rsqrt(total * jnp.float32(1.0 / N_GLOBAL) + jnp.float32(EPS))
        out_ref[:, :] = y * lax.transpose(inv_rms, (1, 0))

        for rdma in sends:
            rdma.wait_send()

    return pl.pallas_call(
        body,
        out_shape=jax.ShapeDtypeStruct((m, n_per), jnp.float32),
        in_specs=[
            pl.BlockSpec(memory_space=pltpu.VMEM),
            pl.BlockSpec(memory_space=pltpu.VMEM),
        ],
        out_specs=pl.BlockSpec(memory_space=pltpu.VMEM),
        scratch_shapes=[
            pltpu.VMEM((1, m), jnp.float32),
            pltpu.VMEM((N_DEV, 1, m), jnp.float32),
            pltpu.SemaphoreType.DMA((N_DEV - 1,)),
            pltpu.SemaphoreType.DMA((N_DEV,)),
        ],
        compiler_params=pltpu.CompilerParams(collective_id=0),
    )(x, gamma2)
